# baseline (speedup 1.0000x reference)
import functools

import jax
import jax.numpy as jnp
from jax import lax
from jax.experimental import pallas as pl
from jax.experimental.pallas import tpu as pltpu
from jax.experimental.pallas import tpu_sc as plsc

N = 10000
D = 128
E = 320000
NC, NS = 2, 16
DW = D // NC
CHUNK = 128
CPT = 157
EPT = CPT * CHUNK
R = 10240
RPT = R // NS
ZROWS = 32


def _agg_body(with_cnt, src_hbm, dst_hbm, x_hbm, *refs):
  if with_cnt:
    agg_hbm, cnt_hbm = refs[0], refs[1]
    refs = refs[2:]
  else:
    agg_hbm = refs[0]
    refs = refs[1:]
  (src_v, dst_v, rows0_v, rows1_v, rows2_v, zrow_v, ones_v, zcnt_v, acc_sh,
   cnt_sh, sem0, sem1, sem2, ssem0, ssem1, ssem2) = refs
  bufs = (rows0_v, rows1_v, rows2_v)
  sems = (sem0, sem1, sem2)
  ssems = (ssem0, ssem1, ssem2)

  c = lax.axis_index("c")
  s = lax.axis_index("s")
  base = s * RPT

  zero16 = jnp.zeros((16,), jnp.float32)

  def zfill(i, _):
    for j in range(DW // 16):
      zrow_v[i, pl.ds(j * 16, 16)] = zero16
    return 0

  lax.fori_loop(0, ZROWS, zfill, 0)

  def zcopy(i, _):
    pltpu.sync_copy(zrow_v, acc_sh.at[pl.ds(base + i * ZROWS, ZROWS)])
    return 0

  lax.fori_loop(0, RPT // ZROWS, zcopy, 0)

  if with_cnt:
    def zcntfill(i, _):
      zcnt_v[pl.ds(i * 16, 16)] = zero16
      return 0

    lax.fori_loop(0, RPT // 16, zcntfill, 0)
    pltpu.sync_copy(zcnt_v, cnt_sh.at[pl.ds(base, RPT)])
    one16 = jnp.ones((16,), jnp.float32)
    for i in range(CHUNK // 16):
      ones_v[pl.ds(i * 16, 16)] = one16

  plsc.subcore_barrier()

  pltpu.sync_copy(src_hbm.at[c, s], src_v)
  pltpu.sync_copy(dst_hbm.at[s], dst_v)

  def gather(j, b):
    pltpu.async_copy(x_hbm.at[src_v.at[j]], bufs[b], sems[b])

  def wait_scat(j, b):
    pltpu.make_async_copy(bufs[b], acc_sh.at[dst_v.at[j]], ssems[b]).wait()

  def step(j, b, prefetch, wait_prev):
    bn = (b + 2) % 3
    if prefetch:
      if wait_prev:
        wait_scat(j - 1, bn)
      gather(j + 2, bn)
    pltpu.make_async_copy(x_hbm.at[src_v.at[j]], bufs[b], sems[b]).wait()
    pltpu.async_copy(bufs[b], acc_sh.at[dst_v.at[j]], ssems[b], add=True)
    if with_cnt:
      pltpu.sync_copy(ones_v, cnt_sh.at[dst_v.at[j]], add=True)

  gather(0, 0)
  gather(1, 1)
  step(0, 0, True, False)
  step(1, 1, True, True)
  step(2, 2, True, True)

  def triple(i, _):
    j = 3 * i
    step(j, 0, True, True)
    step(j + 1, 1, True, True)
    step(j + 2, 2, True, True)
    return 0

  ntrip = (CPT - 4) // 3
  lax.fori_loop(1, ntrip, triple, 0)
  for j in range(3 * ntrip, CPT):
    step(j, j % 3, j + 2 < CPT, j + 2 < CPT)
  for j in range(CPT - 3, CPT):
    wait_scat(j, j % 3)

  plsc.subcore_barrier()

  pltpu.sync_copy(acc_sh.at[pl.ds(base, RPT)], agg_hbm.at[c, pl.ds(base, RPT)])
  if with_cnt:
    pltpu.sync_copy(cnt_sh.at[pl.ds(base, RPT)], cnt_hbm.at[c, pl.ds(base, RPT)])


def _make_agg(with_cnt):
  mesh = plsc.VectorSubcoreMesh(core_axis_name="c", subcore_axis_name="s")
  out_type = [jax.ShapeDtypeStruct((NC, R, DW), jnp.float32)]
  if with_cnt:
    out_type.append(jax.ShapeDtypeStruct((NC, R), jnp.float32))
  scratch = [
      pltpu.VMEM((CPT, CHUNK), jnp.int32),
      pltpu.VMEM((CPT, CHUNK), jnp.int32),
      pltpu.VMEM((CHUNK, DW), jnp.float32),
      pltpu.VMEM((CHUNK, DW), jnp.float32),
      pltpu.VMEM((CHUNK, DW), jnp.float32),
      pltpu.VMEM((ZROWS, DW), jnp.float32),
      pltpu.VMEM((CHUNK,), jnp.float32),
      pltpu.VMEM((RPT,), jnp.float32),
      pltpu.VMEM_SHARED((R, DW), jnp.float32),
      pltpu.VMEM_SHARED((R,), jnp.float32),
      pltpu.SemaphoreType.DMA,
      pltpu.SemaphoreType.DMA,
      pltpu.SemaphoreType.DMA,
      pltpu.SemaphoreType.DMA,
      pltpu.SemaphoreType.DMA,
      pltpu.SemaphoreType.DMA,
  ]
  return pl.kernel(
      functools.partial(_agg_body, with_cnt),
      out_type=tuple(out_type) if with_cnt else out_type[0],
      mesh=mesh,
      scratch_types=scratch,
      compiler_params=pltpu.CompilerParams(use_tc_tiling_on_sc=False),
  )


def _linear_body(relu, split_out, aggp_ref, cntp_ref, xp_ref, wl_ref, wr_ref,
                 b_ref, o_ref):
  recip = 1.0 / jnp.clip(cntp_ref[0], 1.0, None)
  y = (
      jnp.dot(aggp_ref[0] * recip, wl_ref[:DW],
              preferred_element_type=jnp.float32)
      + jnp.dot(aggp_ref[1] * recip, wl_ref[DW:],
                preferred_element_type=jnp.float32)
      + jnp.dot(xp_ref[0], wr_ref[:DW], preferred_element_type=jnp.float32)
      + jnp.dot(xp_ref[1], wr_ref[DW:], preferred_element_type=jnp.float32)
      + b_ref[...]
  )
  if relu:
    y = jnp.maximum(y, 0.0)
  if split_out:
    o_ref[0] = y[:, :DW]
    o_ref[1] = y[:, DW:]
  else:
    o_ref[...] = y


def _linear(relu, split_out, agg_p, cnt_p, x_p, wl, wr, b):
  br = 1000
  if split_out:
    out_shape = jax.ShapeDtypeStruct((NC, N, DW), jnp.float32)
    out_spec = pl.BlockSpec((NC, br, DW), lambda i: (0, i, 0))
  else:
    out_shape = jax.ShapeDtypeStruct((N, D), jnp.float32)
    out_spec = pl.BlockSpec((br, D), lambda i: (i, 0))
  grid = (N // br,)
  return pl.pallas_call(
      functools.partial(_linear_body, relu, split_out),
      grid=grid,
      in_specs=[
          pl.BlockSpec((NC, br, DW), lambda i: (0, i, 0)),
          pl.BlockSpec((NC, br, 1), lambda i: (0, i, 0)),
          pl.BlockSpec((NC, br, DW), lambda i: (0, i, 0)),
          pl.BlockSpec((D, D), lambda i: (0, 0)),
          pl.BlockSpec((D, D), lambda i: (0, 0)),
          pl.BlockSpec((1, D), lambda i: (0, 0)),
      ],
      out_specs=out_spec,
      out_shape=out_shape,
  )(agg_p, cnt_p.reshape(NC, R, 1), x_p, wl, wr, b)


_agg_with_cnt = _make_agg(True)
_agg_no_cnt = _make_agg(False)


def kernel(x, edge_index, W1_l, W1_r, b1, W2_l, W2_r, b2):
  src = edge_index[0].astype(jnp.int32)
  dst = edge_index[1].astype(jnp.int32)
  ept_real = E // NS
  pad = EPT - ept_real
  pad_dst = jnp.broadcast_to(N + (jnp.arange(pad) % (R - N)), (NS, pad))
  src_p = jnp.concatenate(
      [src.reshape(NS, ept_real), jnp.zeros((NS, pad), jnp.int32)], axis=1
  ).reshape(NS, CPT, CHUNK)
  src_p = jnp.stack([src_p, src_p + N])
  dst_p = jnp.concatenate(
      [dst.reshape(NS, ept_real), pad_dst.astype(jnp.int32)], axis=1
  ).reshape(NS, CPT, CHUNK)
  x_s = jnp.stack([x[:, :DW], x[:, DW:]])

  agg1, cnt = _agg_with_cnt(src_p, dst_p, x_s.reshape(NC * N, DW))
  h_s = _linear(True, True, agg1, cnt, x_s, W1_l, W1_r, b1.reshape(1, D))
  agg2 = _agg_no_cnt(src_p, dst_p, h_s.reshape(NC * N, DW))
  return _linear(False, False, agg2, cnt, h_s, W2_l, W2_r, b2.reshape(1, D))


# speedup vs baseline: 1.2335x; 1.2335x over previous
import functools

import jax
import jax.numpy as jnp
from jax import lax
from jax.experimental import pallas as pl
from jax.experimental.pallas import tpu as pltpu
from jax.experimental.pallas import tpu_sc as plsc

N = 10000
D = 128
E = 320000
NC, NS = 2, 16
DW = D // NC
CHUNK = 128
CPT = 157
EPT = CPT * CHUNK
R = 10240
RPT = R // NS
ZROWS = 32


def _agg_body(with_cnt, src_hbm, dst_hbm, x_hbm, *refs):
  if with_cnt:
    agg_hbm, cnt_hbm = refs[0], refs[1]
    refs = refs[2:]
  else:
    agg_hbm = refs[0]
    refs = refs[1:]
  (src_v, dst_v, rows0_v, rows1_v, rows2_v, zrow_v, ones_v, zcnt_v, acc_sh,
   cnt_sh, sem0, sem1, sem2, ssem0, ssem1, ssem2) = refs
  bufs = (rows0_v, rows1_v, rows2_v)
  sems = (sem0, sem1, sem2)
  ssems = (ssem0, ssem1, ssem2)

  c = lax.axis_index("c")
  s = lax.axis_index("s")
  base = s * RPT

  zero16 = jnp.zeros((16,), jnp.float32)
  zero32b = jnp.zeros((32,), jnp.bfloat16)

  def zfill(i, _):
    for j in range(DW // 32):
      zrow_v[i, pl.ds(j * 32, 32)] = zero32b
    return 0

  lax.fori_loop(0, ZROWS, zfill, 0)

  def zcopy(i, _):
    pltpu.sync_copy(zrow_v, acc_sh.at[pl.ds(base + i * ZROWS, ZROWS)])
    return 0

  lax.fori_loop(0, RPT // ZROWS, zcopy, 0)

  if with_cnt:
    def zcntfill(i, _):
      zcnt_v[pl.ds(i * 16, 16)] = zero16
      return 0

    lax.fori_loop(0, RPT // 16, zcntfill, 0)
    pltpu.sync_copy(zcnt_v, cnt_sh.at[pl.ds(base, RPT)])
    one16 = jnp.ones((16,), jnp.float32)
    for i in range(CHUNK // 16):
      ones_v[pl.ds(i * 16, 16)] = one16

  plsc.subcore_barrier()

  pltpu.sync_copy(src_hbm.at[c, s], src_v)
  pltpu.sync_copy(dst_hbm.at[s], dst_v)

  def gather(j, b):
    pltpu.async_copy(x_hbm.at[src_v.at[j]], bufs[b], sems[b])

  def wait_scat(j, b):
    pltpu.make_async_copy(bufs[b], acc_sh.at[dst_v.at[j]], ssems[b]).wait()

  def step(j, b, prefetch, wait_prev):
    bn = (b + 2) % 3
    if prefetch:
      if wait_prev:
        wait_scat(j - 1, bn)
      gather(j + 2, bn)
    pltpu.make_async_copy(x_hbm.at[src_v.at[j]], bufs[b], sems[b]).wait()
    pltpu.async_copy(bufs[b], acc_sh.at[dst_v.at[j]], ssems[b], add=True)
    if with_cnt:
      pltpu.sync_copy(ones_v, cnt_sh.at[dst_v.at[j]], add=True)

  gather(0, 0)
  gather(1, 1)
  step(0, 0, True, False)
  step(1, 1, True, True)
  step(2, 2, True, True)

  def triple(i, _):
    j = 3 * i
    step(j, 0, True, True)
    step(j + 1, 1, True, True)
    step(j + 2, 2, True, True)
    return 0

  ntrip = (CPT - 4) // 3
  lax.fori_loop(1, ntrip, triple, 0)
  for j in range(3 * ntrip, CPT):
    step(j, j % 3, j + 2 < CPT, j + 2 < CPT)
  for j in range(CPT - 3, CPT):
    wait_scat(j, j % 3)

  plsc.subcore_barrier()

  pltpu.sync_copy(acc_sh.at[pl.ds(base, RPT)], agg_hbm.at[c, pl.ds(base, RPT)])
  if with_cnt:
    pltpu.sync_copy(cnt_sh.at[pl.ds(base, RPT)], cnt_hbm.at[c, pl.ds(base, RPT)])


def _make_agg(with_cnt):
  mesh = plsc.VectorSubcoreMesh(core_axis_name="c", subcore_axis_name="s")
  out_type = [jax.ShapeDtypeStruct((NC, R, DW), jnp.bfloat16)]
  if with_cnt:
    out_type.append(jax.ShapeDtypeStruct((NC, R), jnp.float32))
  scratch = [
      pltpu.VMEM((CPT, CHUNK), jnp.int32),
      pltpu.VMEM((CPT, CHUNK), jnp.int32),
      pltpu.VMEM((CHUNK, DW), jnp.bfloat16),
      pltpu.VMEM((CHUNK, DW), jnp.bfloat16),
      pltpu.VMEM((CHUNK, DW), jnp.bfloat16),
      pltpu.VMEM((ZROWS, DW), jnp.bfloat16),
      pltpu.VMEM((CHUNK,), jnp.float32),
      pltpu.VMEM((RPT,), jnp.float32),
      pltpu.VMEM_SHARED((R, DW), jnp.bfloat16),
      pltpu.VMEM_SHARED((R,), jnp.float32),
      pltpu.SemaphoreType.DMA,
      pltpu.SemaphoreType.DMA,
      pltpu.SemaphoreType.DMA,
      pltpu.SemaphoreType.DMA,
      pltpu.SemaphoreType.DMA,
      pltpu.SemaphoreType.DMA,
  ]
  return pl.kernel(
      functools.partial(_agg_body, with_cnt),
      out_type=tuple(out_type) if with_cnt else out_type[0],
      mesh=mesh,
      scratch_types=scratch,
      compiler_params=pltpu.CompilerParams(use_tc_tiling_on_sc=False),
  )


def _linear_body(relu, split_out, aggp_ref, cntp_ref, xp_ref, wl_ref, wr_ref,
                 b_ref, *o_refs):
  recip = 1.0 / jnp.clip(cntp_ref[0], 1.0, None)
  y = (
      jnp.dot(aggp_ref[0].astype(jnp.float32) * recip, wl_ref[:DW],
              preferred_element_type=jnp.float32)
      + jnp.dot(aggp_ref[1].astype(jnp.float32) * recip, wl_ref[DW:],
                preferred_element_type=jnp.float32)
      + jnp.dot(xp_ref[0], wr_ref[:DW], preferred_element_type=jnp.float32)
      + jnp.dot(xp_ref[1], wr_ref[DW:], preferred_element_type=jnp.float32)
      + b_ref[...]
  )
  if relu:
    y = jnp.maximum(y, 0.0)
  if split_out:
    o_ref, obf_ref = o_refs
    o_ref[0] = y[:, :DW]
    o_ref[1] = y[:, DW:]
    yb = y.astype(jnp.bfloat16)
    obf_ref[0] = yb[:, :DW]
    obf_ref[1] = yb[:, DW:]
  else:
    o_refs[0][...] = y


def _linear(relu, split_out, agg_p, cnt_p, x_p, wl, wr, b):
  br = 1000
  if split_out:
    out_shape = (jax.ShapeDtypeStruct((NC, N, DW), jnp.float32),
                 jax.ShapeDtypeStruct((NC, N, DW), jnp.bfloat16))
    out_spec = (pl.BlockSpec((NC, br, DW), lambda i: (0, i, 0)),
                pl.BlockSpec((NC, br, DW), lambda i: (0, i, 0)))
  else:
    out_shape = jax.ShapeDtypeStruct((N, D), jnp.float32)
    out_spec = pl.BlockSpec((br, D), lambda i: (i, 0))
  grid = (N // br,)
  return pl.pallas_call(
      functools.partial(_linear_body, relu, split_out),
      grid=grid,
      in_specs=[
          pl.BlockSpec((NC, br, DW), lambda i: (0, i, 0)),
          pl.BlockSpec((NC, br, 1), lambda i: (0, i, 0)),
          pl.BlockSpec((NC, br, DW), lambda i: (0, i, 0)),
          pl.BlockSpec((D, D), lambda i: (0, 0)),
          pl.BlockSpec((D, D), lambda i: (0, 0)),
          pl.BlockSpec((1, D), lambda i: (0, 0)),
      ],
      out_specs=out_spec,
      out_shape=out_shape,
  )(agg_p, cnt_p.reshape(NC, R, 1), x_p, wl, wr, b)


_agg_with_cnt = _make_agg(True)
_agg_no_cnt = _make_agg(False)


def kernel(x, edge_index, W1_l, W1_r, b1, W2_l, W2_r, b2):
  src = edge_index[0].astype(jnp.int32)
  dst = edge_index[1].astype(jnp.int32)
  ept_real = E // NS
  pad = EPT - ept_real
  pad_dst = jnp.broadcast_to(N + (jnp.arange(pad) % (R - N)), (NS, pad))
  src_p = jnp.concatenate(
      [src.reshape(NS, ept_real), jnp.zeros((NS, pad), jnp.int32)], axis=1
  ).reshape(NS, CPT, CHUNK)
  src_p = jnp.stack([src_p, src_p + N])
  dst_p = jnp.concatenate(
      [dst.reshape(NS, ept_real), pad_dst.astype(jnp.int32)], axis=1
  ).reshape(NS, CPT, CHUNK)
  x_s = jnp.stack([x[:, :DW], x[:, DW:]])

  agg1, cnt = _agg_with_cnt(
      src_p, dst_p, x_s.astype(jnp.bfloat16).reshape(NC * N, DW))
  h_s, h_bf = _linear(True, True, agg1, cnt, x_s, W1_l, W1_r, b1.reshape(1, D))
  agg2 = _agg_no_cnt(src_p, dst_p, h_bf.reshape(NC * N, DW))
  return _linear(False, False, agg2, cnt, h_s, W2_l, W2_r, b2.reshape(1, D))
